# baseline (speedup 1.0000x reference)
import functools

import jax
import jax.numpy as jnp
from jax import lax
from jax.experimental import pallas as pl
from jax.experimental.pallas import tpu as pltpu
from jax.experimental.pallas import tpu_sc as plsc

B = 1024
N_CHANS = 64
WINDOW = 1000
N_OUT = 4
E = 16
FEAT = 512

WB = 40
NSTEP = WINDOW // WB
BB2 = 256


def _tc1_body(xT_ref, m_ref, acc_ref):
    i = pl.program_id(0)

    @pl.when(i == 0)
    def _():
        acc_ref[...] = jnp.zeros_like(acc_ref)

    acc_ref[...] += jnp.sum(xT_ref[...], axis=1)

    @pl.when(i == NSTEP - 1)
    def _():
        m_ref[...] = acc_ref[...] * (1.0 / WINDOW)


def _sc_gather_body(table_hbm, sid_hbm, wt_hbm, idx_v, rows_v, sem, nc):
    bpw = idx_v.shape[0]
    wid = lax.axis_index("s") * nc + lax.axis_index("c")
    base = wid * bpw
    pltpu.sync_copy(sid_hbm.at[pl.ds(base, bpw)], idx_v)
    pltpu.async_copy(table_hbm.at[idx_v], rows_v, sem).wait()
    pltpu.sync_copy(rows_v, wt_hbm.at[pl.ds(base, bpw)])


def _tc2_body(m_ref, sid_ref, Wbb_ref, bbb_ref, bh_ref, wt_ref, out_ref):
    dn = (((0,), (0,)), ((), ()))
    feats = jax.lax.dot_general(m_ref[...], Wbb_ref[...], dn,
                                preferred_element_type=jnp.float32)
    feats = jnp.maximum(feats + bbb_ref[...], 0.0)
    cols = []
    for o in range(N_OUT):
        cols.append(jnp.sum(feats * wt_ref[:, o, :], axis=1, keepdims=True))
    outv = jnp.concatenate(cols, axis=1)
    sid = sid_ref[...]
    onehot = (jax.lax.broadcasted_iota(jnp.int32, (BB2, E), 1)
              == sid).astype(jnp.float32)
    bias = jnp.dot(onehot, bh_ref[...], preferred_element_type=jnp.float32)
    out_ref[...] = outv + bias


@jax.jit
def kernel(x, subject_ids, W_bb, b_bb, W_heads, b_heads):
    xT = jnp.transpose(x, (1, 2, 0))
    sid = subject_ids.astype(jnp.int32)
    table = W_heads.transpose(0, 2, 1).reshape(E, N_OUT * FEAT)
    bbb = b_bb.reshape(1, FEAT)

    info = plsc.get_sparse_core_info()
    nc, ns = info.num_cores, info.num_subcores
    mesh = plsc.VectorSubcoreMesh(core_axis_name="c", subcore_axis_name="s")
    bpw = B // (nc * ns)
    sc_gather = pl.kernel(
        functools.partial(_sc_gather_body, nc=nc),
        mesh=mesh,
        compiler_params=pltpu.CompilerParams(use_tc_tiling_on_sc=False,
                                             needs_layout_passes=False),
        out_type=jax.ShapeDtypeStruct((B, N_OUT * FEAT), jnp.float32),
        scratch_types=[
            pltpu.VMEM((bpw,), jnp.int32),
            pltpu.VMEM((bpw, N_OUT * FEAT), jnp.float32),
            pltpu.SemaphoreType.DMA,
        ],
    )
    wt = sc_gather(table, sid)
    wt3 = wt.reshape(B, N_OUT, FEAT)

    m = pl.pallas_call(
        _tc1_body,
        grid=(NSTEP,),
        in_specs=[pl.BlockSpec((N_CHANS, WB, B), lambda i: (0, i, 0))],
        out_specs=pl.BlockSpec((N_CHANS, B), lambda i: (0, 0)),
        out_shape=jax.ShapeDtypeStruct((N_CHANS, B), jnp.float32),
        scratch_shapes=[pltpu.VMEM((N_CHANS, B), jnp.float32)],
    )(xT)

    out = pl.pallas_call(
        _tc2_body,
        grid=(B // BB2,),
        in_specs=[
            pl.BlockSpec((N_CHANS, BB2), lambda j: (0, j)),
            pl.BlockSpec((BB2, 1), lambda j: (j, 0)),
            pl.BlockSpec((N_CHANS, FEAT), lambda j: (0, 0)),
            pl.BlockSpec((1, FEAT), lambda j: (0, 0)),
            pl.BlockSpec((E, N_OUT), lambda j: (0, 0)),
            pl.BlockSpec((BB2, N_OUT, FEAT), lambda j: (j, 0, 0)),
        ],
        out_specs=pl.BlockSpec((BB2, N_OUT), lambda j: (j, 0)),
        out_shape=jax.ShapeDtypeStruct((B, N_OUT), jnp.float32),
    )(m, sid.reshape(B, 1), W_bb, bbb, b_heads, wt3)
    return out
